# baseline (speedup 1.0000x reference)
import jax
import jax.numpy as jnp
from jax.experimental import pallas as pl
from jax.experimental.pallas import tpu as pltpu

_H = 384
_W = 384
_N = float(_H * _W)
_R = _H // 8
_CPB = 8
_NB = 96 // _CPB


def _fold3(a):
    return jnp.maximum(jnp.maximum(a[:, :128], a[:, 128:256]), a[:, 256:])


def _fold3s(a):
    return a[:, :128] + a[:, 128:256] + a[:, 256:]


def _vscalar(a):
    return jnp.sum(jnp.sum(a, axis=1, keepdims=True), axis=0,
                   keepdims=True)


def _step(i, x_ref, m2_ref, m2g_ref, cef_ref, out_ref,
          xsA_ref, wsvA_ref, accA_ref, xsB_ref, wsvB_ref, accB_ref):
    def tau_step(_, carry):
        out = []
        for c in range(_CPB):
            W, _tau = carry[c]
            mx = jnp.max(W, axis=1, keepdims=True)
            mx = jnp.max(mx, axis=0, keepdims=True)
            mxb = jax.lax.broadcast_in_dim(mx, (8, 128), (0, 1))
            W = jnp.where(W == mxb, -jnp.inf, W)
            out.append((W, mxb))
        return tuple(out)

    Ws = [wsvB_ref[c] for c in range(_CPB)]
    taus_c = jax.lax.fori_loop(
        0, 10, tau_step,
        tuple((Ws[c], Ws[c]) for c in range(_CPB)),
        unroll=True,
    )
    tau_wide = [
        jnp.concatenate([taus_c[c][1]] * (_W // 128), axis=1)
        for c in range(_CPB)
    ]

    sums_prev = [
        (_vscalar(accB_ref[c, 0]), _vscalar(accB_ref[c, 1]),
         _vscalar(accB_ref[c, 2]), _vscalar(accB_ref[c, 3]))
        for c in range(_CPB)
    ]

    def chunk(j, carry):
        main_c, filt_c = carry
        m2c = m2_ref[0, j]
        m2gc = m2g_ref[0, j]
        mains = []
        filts = []
        for c in range(_CPB):
            aS, aA, aB, aD, aM = main_c[c]
            xv = x_ref[c, j]
            xsA_ref[c, j] = xv
            vm2 = xv * m2c
            aS = aS + _fold3s(xv)
            aA = aA + _fold3s(xv * vm2)
            aB = aB + _fold3s(vm2)
            aD = aD + _fold3s(xv * m2gc)
            aM = jnp.maximum(aM, xv)
            mains.append((aS, aA, aB, aD, aM))

            aC, aV = filt_c[c]
            pv = xsB_ref[c, j]
            sel = pv >= tau_wide[c]
            aC = aC + _fold3s(jnp.where(sel, 1.0, 0.0))
            aV = aV + _fold3s(jnp.where(sel, pv, 0.0))
            filts.append((aC, aV))
        return tuple(mains), tuple(filts)

    zero1 = jnp.zeros((8, 128), jnp.float32)
    main_init = tuple(
        (zero1, zero1, zero1, zero1,
         jnp.full((8, _W), -jnp.inf, jnp.float32))
        for _ in range(_CPB)
    )
    filt_init = tuple((zero1, zero1) for _ in range(_CPB))
    mains, filts = jax.lax.fori_loop(
        0, _R, chunk, (main_init, filt_init), unroll=True)

    for c in range(_CPB):
        aS, aA, aB, aD, M = mains[c]
        wsvA_ref[c] = _fold3(M)
        accA_ref[c, 0] = aS
        accA_ref[c, 1] = aA
        accA_ref[c, 2] = aB
        accA_ref[c, 3] = aD

    C = cef_ref[0, 0:1, 0:1]
    E = cef_ref[0, 1:2, 0:1]
    F = cef_ref[0, 2:3, 0:1]

    loss = jnp.zeros((1, 1), jnp.float32)
    mismatch = jnp.zeros((1, 1), jnp.float32)
    for c in range(_CPB):
        cnt = _vscalar(filts[c][0])
        ssum = _vscalar(filts[c][1])
        mismatch = mismatch + jnp.abs(cnt - 10.0)
        S, A, B, D = sums_prev[c]
        mu = S / _N
        max_avg = ssum / 10.0 - mu
        denom = jnp.where(max_avg < 1e-20, max_avg + 1e-19, max_avg)
        num = ((A - 2.0 * mu * B + mu * mu * C) / denom
               - 2.0 * (D - mu * E)) / denom + F
        loss = loss + num / _N

    gate = jnp.where(i > 0, 1.0, 0.0)
    loss = jnp.where(gate > 0.0, loss, 0.0)
    mismatch = jnp.where(gate > 0.0, mismatch, 0.0)
    upd = jnp.concatenate([
        jax.lax.broadcast_in_dim(loss, (1, 8, 128), (1, 2)),
        jax.lax.broadcast_in_dim(mismatch, (1, 8, 128), (1, 2)),
    ], axis=0)
    out_ref[...] += upd


def _body(x_ref, gt_ref, m_ref, out_ref,
          m2_ref, m2g_ref, cef_ref,
          xs0_ref, xs1_ref, wsv0_ref, wsv1_ref, acc0_ref, acc1_ref):
    i = pl.program_id(0)
    par = jax.lax.rem(i, 2)

    @pl.when(i == 0)
    def _():
        m = m_ref[0]
        g = gt_ref[0]
        m2 = m * m
        m2g = m2 * g
        m2_ref[0] = m2
        m2g_ref[0] = m2g
        cef_ref[...] = jnp.concatenate([
            jax.lax.broadcast_in_dim(_vscalar2(m2), (1, 1, 128), (1, 2)),
            jax.lax.broadcast_in_dim(_vscalar2(m2g), (1, 1, 128), (1, 2)),
            jax.lax.broadcast_in_dim(_vscalar2(m2g * g), (1, 1, 128),
                                     (1, 2)),
        ], axis=1)
        out_ref[...] = jnp.zeros(out_ref.shape, jnp.float32)

    @pl.when(par == 0)
    def _():
        _step(i, x_ref, m2_ref, m2g_ref, cef_ref, out_ref,
              xs0_ref, wsv0_ref, acc0_ref, xs1_ref, wsv1_ref, acc1_ref)

    @pl.when(par == 1)
    def _():
        _step(i, x_ref, m2_ref, m2g_ref, cef_ref, out_ref,
              xs1_ref, wsv1_ref, acc1_ref, xs0_ref, wsv0_ref, acc0_ref)


def _vscalar2(a):
    s = jnp.sum(a, axis=0)
    s = _fold3s(s)
    return _vscalar(s)


def _fast(pattern, pattern_gt, mask):
    ch = pattern.shape[1]
    x = pattern.reshape(ch, _R, 8, _W)
    out = pl.pallas_call(
        _body,
        grid=(_NB + 1,),
        in_specs=[
            pl.BlockSpec((_CPB, _R, 8, _W),
                         lambda i: (jnp.minimum(i, _NB - 1), 0, 0, 0)),
            pl.BlockSpec((1, _R, 8, _W), lambda i: (0, 0, 0, 0)),
            pl.BlockSpec((1, _R, 8, _W), lambda i: (0, 0, 0, 0)),
        ],
        out_specs=pl.BlockSpec((2, 8, 128), lambda i: (0, 0, 0)),
        out_shape=jax.ShapeDtypeStruct((2, 8, 128), jnp.float32),
        scratch_shapes=[
            pltpu.VMEM((1, _R, 8, _W), jnp.float32),
            pltpu.VMEM((1, _R, 8, _W), jnp.float32),
            pltpu.VMEM((1, 3, 128), jnp.float32),
            pltpu.VMEM((_CPB, _R, 8, _W), jnp.float32),
            pltpu.VMEM((_CPB, _R, 8, _W), jnp.float32),
            pltpu.VMEM((_CPB, 8, 128), jnp.float32),
            pltpu.VMEM((_CPB, 8, 128), jnp.float32),
            pltpu.VMEM((_CPB, 4, 8, 128), jnp.float32),
            pltpu.VMEM((_CPB, 4, 8, 128), jnp.float32),
        ],
    )(x, pattern_gt.reshape(1, _R, 8, _W), mask.reshape(1, _R, 8, _W))
    return out[0, 0, 0].reshape(1), out[1, 0, 0]


def _exact_body(x_ref, gt_ref, m_ref, out_ref, m2_ref, m2g_ref, cef_ref):
    i = pl.program_id(0)

    @pl.when(i == 0)
    def _():
        m = m_ref[0]
        g = gt_ref[0]
        m2 = m * m
        m2g = m2 * g
        m2_ref[0] = m2
        m2g_ref[0] = m2g
        cef_ref[0] = jnp.sum(m2)
        cef_ref[1] = jnp.sum(m2g)
        cef_ref[2] = jnp.sum(m2g * g)
        out_ref[...] = jnp.zeros(out_ref.shape, jnp.float32)

    def chunk(j, carry):
        aS, aA, aB, aD, aM = carry
        xv = x_ref[0, j]
        m2c = m2_ref[0, j]
        m2gc = m2g_ref[0, j]
        vm2 = xv * m2c
        return (aS + xv, aA + xv * vm2, aB + vm2, aD + xv * m2gc,
                jnp.maximum(aM, xv))

    zero = jnp.zeros((8, _W), jnp.float32)
    aS, aA, aB, aD, M = jax.lax.fori_loop(
        0, _R, chunk,
        (zero, zero, zero, zero, jnp.full((8, _W), -jnp.inf, jnp.float32)),
        unroll=4,
    )
    S = jnp.sum(aS)
    A = jnp.sum(aA)
    B = jnp.sum(aB)
    D = jnp.sum(aD)

    def tau_step(_, carry):
        MM, tau = carry
        mx = jnp.max(MM)
        MM = jnp.where(MM == mx, -jnp.inf, MM)
        return MM, mx

    _, tau = jax.lax.fori_loop(0, 10, tau_step, (M, jnp.float32(-jnp.inf)))

    def fchunk(j, carry):
        aC, aV = carry
        xv = x_ref[0, j]
        sel = xv >= tau
        return (aC + jnp.where(sel, 1.0, 0.0),
                aV + jnp.where(sel, xv, 0.0))

    aC, aV = jax.lax.fori_loop(0, _R, fchunk, (zero, zero), unroll=4)
    cnt = jnp.sum(aC)
    ssum = jnp.sum(aV)

    def exact_fallback(_):
        def step(_, carry):
            bound, acc, rem = carry
            v = x_ref[0]
            w = jnp.where((v >= tau) & (v < bound), v, -jnp.inf)
            mx = jnp.max(w)
            cc = jnp.sum(jnp.where(w == mx, 1.0, 0.0))
            take = jnp.minimum(cc, rem)
            acc = acc + jnp.where(take > 0.0, take * mx, 0.0)
            rem = rem - take
            return mx, acc, rem

        _, acc, _ = jax.lax.fori_loop(
            0, 10, step, (jnp.float32(jnp.inf), jnp.float32(0.0),
                          jnp.float32(10.0))
        )
        return acc

    top10_sum = jax.lax.cond(cnt == 10.0, lambda _: ssum, exact_fallback,
                             operand=None)

    C = cef_ref[0]
    E = cef_ref[1]
    F = cef_ref[2]

    mu = S / _N
    max_avg = top10_sum / 10.0 - mu
    denom = jnp.where(max_avg < 1e-20, max_avg + 1e-19, max_avg)
    num = ((A - 2.0 * mu * B + mu * mu * C) / denom
           - 2.0 * (D - mu * E)) / denom + F
    loss_i = num / _N

    out_ref[...] += jnp.full(out_ref.shape, loss_i, dtype=jnp.float32)


def _exact(pattern, pattern_gt, mask):
    ch = pattern.shape[1]
    x = pattern.reshape(ch, _R, 8, _W)
    out = pl.pallas_call(
        _exact_body,
        grid=(ch,),
        in_specs=[
            pl.BlockSpec((1, _R, 8, _W), lambda i: (i, 0, 0, 0)),
            pl.BlockSpec((1, _R, 8, _W), lambda i: (0, 0, 0, 0)),
            pl.BlockSpec((1, _R, 8, _W), lambda i: (0, 0, 0, 0)),
        ],
        out_specs=pl.BlockSpec((8, 128), lambda i: (0, 0)),
        out_shape=jax.ShapeDtypeStruct((8, 128), jnp.float32),
        scratch_shapes=[
            pltpu.VMEM((1, _R, 8, _W), jnp.float32),
            pltpu.VMEM((1, _R, 8, _W), jnp.float32),
            pltpu.SMEM((3,), jnp.float32),
        ],
    )(x, pattern_gt.reshape(1, _R, 8, _W), mask.reshape(1, _R, 8, _W))
    return out[0, 0].reshape(1)


@jax.jit
def kernel(pattern, pattern_gt, mask):
    loss_fast, mismatch = _fast(pattern, pattern_gt, mask)
    return jax.lax.cond(
        mismatch == 0.0,
        lambda _: loss_fast,
        lambda _: _exact(pattern, pattern_gt, mask),
        operand=None,
    )


# speedup vs baseline: 4.7630x; 4.7630x over previous
import jax
import jax.numpy as jnp
from jax.experimental import pallas as pl
from jax.experimental.pallas import tpu as pltpu

_H = 384
_W = 384
_N = float(_H * _W)
_R = _H // 8
_CPB = 8
_NB = 96 // _CPB


def _fold3(a):
    return jnp.maximum(jnp.maximum(a[:, :128], a[:, 128:256]), a[:, 256:])


def _fold3s(a):
    return a[:, :128] + a[:, 128:256] + a[:, 256:]


def _vscalar(a):
    return jnp.sum(jnp.sum(a, axis=1, keepdims=True), axis=0,
                   keepdims=True)


def _vscalarw(a):
    return jnp.sum(jnp.sum(a, axis=1, keepdims=True), axis=0,
                   keepdims=True)


def _step(i, x_ref, m2_ref, m2g_ref, cef_ref, out_ref,
          xsA_ref, wsvA_ref, accA_ref, xsB_ref, wsvB_ref, accB_ref):
    def tau_step(_, carry):
        out = []
        for c in range(_CPB):
            Ma, Mb, tau, rem = carry[c]
            mx = jnp.maximum(jnp.max(Ma, axis=1, keepdims=True),
                             jnp.max(Mb, axis=1, keepdims=True))
            mx = jnp.max(mx, axis=0, keepdims=True)
            mxb = jax.lax.broadcast_in_dim(mx, (8, _W), (0, 1))
            hita = Ma == mxb
            hitb = Mb == mxb
            cnte = _vscalarw(jnp.where(hita, 1.0, 0.0)
                             + jnp.where(hitb, 1.0, 0.0))
            tau = jnp.where(rem > 0.0, mx, tau)
            rem = rem - cnte
            Ma = jnp.where(hita, -jnp.inf, Ma)
            Mb = jnp.where(hitb, -jnp.inf, Mb)
            out.append((Ma, Mb, tau, rem))
        return tuple(out)

    ten = jnp.full((1, 1), 10.0, jnp.float32)
    taus_c = jax.lax.fori_loop(
        0, 10, tau_step,
        tuple((wsvB_ref[c, 0], wsvB_ref[c, 1],
               jnp.full((1, 1), -jnp.inf, jnp.float32), ten)
              for c in range(_CPB)),
        unroll=True,
    )
    tau_wide = [
        jax.lax.broadcast_in_dim(taus_c[c][2], (8, _W), (0, 1))
        for c in range(_CPB)
    ]

    sums_prev = [
        (_vscalar(accB_ref[c, 0]), _vscalar(accB_ref[c, 1]),
         _vscalar(accB_ref[c, 2]), _vscalar(accB_ref[c, 3]))
        for c in range(_CPB)
    ]

    def chunk(j, carry):
        main_c, filt_c = carry
        m2c = m2_ref[0, j]
        m2gc = m2g_ref[0, j]
        mains = []
        filts = []
        for c in range(_CPB):
            aS, aA, aB, aD, aM, aM2 = main_c[c]
            xv = x_ref[c, j]
            xsA_ref[c, j] = xv
            vm2 = xv * m2c
            aS = aS + _fold3s(xv)
            aA = aA + _fold3s(xv * vm2)
            aB = aB + _fold3s(vm2)
            aD = aD + _fold3s(xv * m2gc)
            mn = jnp.minimum(aM, xv)
            aM = jnp.maximum(aM, xv)
            aM2 = jnp.maximum(aM2, mn)
            mains.append((aS, aA, aB, aD, aM, aM2))

            aGT, aV, aEQ = filt_c[c]
            pv = xsB_ref[c, j]
            gt = pv > tau_wide[c]
            eq = pv == tau_wide[c]
            aGT = aGT + _fold3s(jnp.where(gt, 1.0, 0.0))
            aV = aV + _fold3s(jnp.where(gt, pv, 0.0))
            aEQ = aEQ + _fold3s(jnp.where(eq, 1.0, 0.0))
            filts.append((aGT, aV, aEQ))
        return tuple(mains), tuple(filts)

    zero1 = jnp.zeros((8, 128), jnp.float32)
    neginf = jnp.full((8, _W), -jnp.inf, jnp.float32)
    main_init = tuple(
        (zero1, zero1, zero1, zero1, neginf, neginf)
        for _ in range(_CPB)
    )
    filt_init = tuple((zero1, zero1, zero1) for _ in range(_CPB))
    mains, filts = jax.lax.fori_loop(
        0, _R, chunk, (main_init, filt_init), unroll=True)

    for c in range(_CPB):
        aS, aA, aB, aD, M, M2 = mains[c]
        wsvA_ref[c, 0] = M
        wsvA_ref[c, 1] = M2
        accA_ref[c, 0] = aS
        accA_ref[c, 1] = aA
        accA_ref[c, 2] = aB
        accA_ref[c, 3] = aD

    C = cef_ref[0, 0:1, 0:1]
    E = cef_ref[0, 1:2, 0:1]
    F = cef_ref[0, 2:3, 0:1]

    loss = jnp.zeros((1, 1), jnp.float32)
    mismatch = jnp.zeros((1, 1), jnp.float32)
    for c in range(_CPB):
        cnt_gt = _vscalar(filts[c][0])
        sum_gt = _vscalar(filts[c][1])
        cnt_eq = _vscalar(filts[c][2])
        tau = taus_c[c][2]
        top10 = sum_gt + (10.0 - cnt_gt) * tau
        dirty = (cnt_gt > 10.0) | ((cnt_gt + cnt_eq) < 10.0)
        mismatch = mismatch + jnp.where(dirty, 1.0, 0.0)
        S, A, B, D = sums_prev[c]
        mu = S / _N
        max_avg = top10 / 10.0 - mu
        denom = jnp.where(max_avg < 1e-20, max_avg + 1e-19, max_avg)
        num = ((A - 2.0 * mu * B + mu * mu * C) / denom
               - 2.0 * (D - mu * E)) / denom + F
        loss = loss + num / _N

    gate = jnp.where(i > 0, 1.0, 0.0)
    loss = jnp.where(gate > 0.0, loss, 0.0)
    mismatch = jnp.where(gate > 0.0, mismatch, 0.0)
    upd = jnp.concatenate([
        jax.lax.broadcast_in_dim(loss, (1, 8, 128), (1, 2)),
        jax.lax.broadcast_in_dim(mismatch, (1, 8, 128), (1, 2)),
    ], axis=0)
    out_ref[...] += upd


def _body(x_ref, gt_ref, m_ref, out_ref,
          m2_ref, m2g_ref, cef_ref,
          xs0_ref, xs1_ref, wsv0_ref, wsv1_ref, acc0_ref, acc1_ref):
    i = pl.program_id(0)
    par = jax.lax.rem(i, 2)

    @pl.when(i == 0)
    def _():
        m = m_ref[0]
        g = gt_ref[0]
        m2 = m * m
        m2g = m2 * g
        m2_ref[0] = m2
        m2g_ref[0] = m2g
        cef_ref[...] = jnp.concatenate([
            jax.lax.broadcast_in_dim(_vscalar2(m2), (1, 1, 128), (1, 2)),
            jax.lax.broadcast_in_dim(_vscalar2(m2g), (1, 1, 128), (1, 2)),
            jax.lax.broadcast_in_dim(_vscalar2(m2g * g), (1, 1, 128),
                                     (1, 2)),
        ], axis=1)
        out_ref[...] = jnp.zeros(out_ref.shape, jnp.float32)

    @pl.when(par == 0)
    def _():
        _step(i, x_ref, m2_ref, m2g_ref, cef_ref, out_ref,
              xs0_ref, wsv0_ref, acc0_ref, xs1_ref, wsv1_ref, acc1_ref)

    @pl.when(par == 1)
    def _():
        _step(i, x_ref, m2_ref, m2g_ref, cef_ref, out_ref,
              xs1_ref, wsv1_ref, acc1_ref, xs0_ref, wsv0_ref, acc0_ref)


def _vscalar2(a):
    s = jnp.sum(a, axis=0)
    s = _fold3s(s)
    return _vscalar(s)


def _fast(pattern, pattern_gt, mask):
    ch = pattern.shape[1]
    x = pattern.reshape(ch, _R, 8, _W)
    out = pl.pallas_call(
        _body,
        grid=(_NB + 1,),
        in_specs=[
            pl.BlockSpec((_CPB, _R, 8, _W),
                         lambda i: (jnp.minimum(i, _NB - 1), 0, 0, 0)),
            pl.BlockSpec((1, _R, 8, _W), lambda i: (0, 0, 0, 0)),
            pl.BlockSpec((1, _R, 8, _W), lambda i: (0, 0, 0, 0)),
        ],
        out_specs=pl.BlockSpec((2, 8, 128), lambda i: (0, 0, 0)),
        out_shape=jax.ShapeDtypeStruct((2, 8, 128), jnp.float32),
        scratch_shapes=[
            pltpu.VMEM((1, _R, 8, _W), jnp.float32),
            pltpu.VMEM((1, _R, 8, _W), jnp.float32),
            pltpu.VMEM((1, 3, 128), jnp.float32),
            pltpu.VMEM((_CPB, _R, 8, _W), jnp.float32),
            pltpu.VMEM((_CPB, _R, 8, _W), jnp.float32),
            pltpu.VMEM((_CPB, 2, 8, _W), jnp.float32),
            pltpu.VMEM((_CPB, 2, 8, _W), jnp.float32),
            pltpu.VMEM((_CPB, 4, 8, 128), jnp.float32),
            pltpu.VMEM((_CPB, 4, 8, 128), jnp.float32),
        ],
    )(x, pattern_gt.reshape(1, _R, 8, _W), mask.reshape(1, _R, 8, _W))
    return out[0, 0, 0].reshape(1), out[1, 0, 0]


def _exact_body(x_ref, gt_ref, m_ref, out_ref, m2_ref, m2g_ref, cef_ref):
    i = pl.program_id(0)

    @pl.when(i == 0)
    def _():
        m = m_ref[0]
        g = gt_ref[0]
        m2 = m * m
        m2g = m2 * g
        m2_ref[0] = m2
        m2g_ref[0] = m2g
        cef_ref[0] = jnp.sum(m2)
        cef_ref[1] = jnp.sum(m2g)
        cef_ref[2] = jnp.sum(m2g * g)
        out_ref[...] = jnp.zeros(out_ref.shape, jnp.float32)

    def chunk(j, carry):
        aS, aA, aB, aD, aM = carry
        xv = x_ref[0, j]
        m2c = m2_ref[0, j]
        m2gc = m2g_ref[0, j]
        vm2 = xv * m2c
        return (aS + xv, aA + xv * vm2, aB + vm2, aD + xv * m2gc,
                jnp.maximum(aM, xv))

    zero = jnp.zeros((8, _W), jnp.float32)
    aS, aA, aB, aD, M = jax.lax.fori_loop(
        0, _R, chunk,
        (zero, zero, zero, zero, jnp.full((8, _W), -jnp.inf, jnp.float32)),
        unroll=4,
    )
    S = jnp.sum(aS)
    A = jnp.sum(aA)
    B = jnp.sum(aB)
    D = jnp.sum(aD)

    def tau_step(_, carry):
        MM, tau = carry
        mx = jnp.max(MM)
        MM = jnp.where(MM == mx, -jnp.inf, MM)
        return MM, mx

    _, tau = jax.lax.fori_loop(0, 10, tau_step, (M, jnp.float32(-jnp.inf)))

    def fchunk(j, carry):
        aC, aV = carry
        xv = x_ref[0, j]
        sel = xv >= tau
        return (aC + jnp.where(sel, 1.0, 0.0),
                aV + jnp.where(sel, xv, 0.0))

    aC, aV = jax.lax.fori_loop(0, _R, fchunk, (zero, zero), unroll=4)
    cnt = jnp.sum(aC)
    ssum = jnp.sum(aV)

    def exact_fallback(_):
        def step(_, carry):
            bound, acc, rem = carry
            v = x_ref[0]
            w = jnp.where((v >= tau) & (v < bound), v, -jnp.inf)
            mx = jnp.max(w)
            cc = jnp.sum(jnp.where(w == mx, 1.0, 0.0))
            take = jnp.minimum(cc, rem)
            acc = acc + jnp.where(take > 0.0, take * mx, 0.0)
            rem = rem - take
            return mx, acc, rem

        _, acc, _ = jax.lax.fori_loop(
            0, 10, step, (jnp.float32(jnp.inf), jnp.float32(0.0),
                          jnp.float32(10.0))
        )
        return acc

    top10_sum = jax.lax.cond(cnt == 10.0, lambda _: ssum, exact_fallback,
                             operand=None)

    C = cef_ref[0]
    E = cef_ref[1]
    F = cef_ref[2]

    mu = S / _N
    max_avg = top10_sum / 10.0 - mu
    denom = jnp.where(max_avg < 1e-20, max_avg + 1e-19, max_avg)
    num = ((A - 2.0 * mu * B + mu * mu * C) / denom
           - 2.0 * (D - mu * E)) / denom + F
    loss_i = num / _N

    out_ref[...] += jnp.full(out_ref.shape, loss_i, dtype=jnp.float32)


def _exact(pattern, pattern_gt, mask):
    ch = pattern.shape[1]
    x = pattern.reshape(ch, _R, 8, _W)
    out = pl.pallas_call(
        _exact_body,
        grid=(ch,),
        in_specs=[
            pl.BlockSpec((1, _R, 8, _W), lambda i: (i, 0, 0, 0)),
            pl.BlockSpec((1, _R, 8, _W), lambda i: (0, 0, 0, 0)),
            pl.BlockSpec((1, _R, 8, _W), lambda i: (0, 0, 0, 0)),
        ],
        out_specs=pl.BlockSpec((8, 128), lambda i: (0, 0)),
        out_shape=jax.ShapeDtypeStruct((8, 128), jnp.float32),
        scratch_shapes=[
            pltpu.VMEM((1, _R, 8, _W), jnp.float32),
            pltpu.VMEM((1, _R, 8, _W), jnp.float32),
            pltpu.SMEM((3,), jnp.float32),
        ],
    )(x, pattern_gt.reshape(1, _R, 8, _W), mask.reshape(1, _R, 8, _W))
    return out[0, 0].reshape(1)


@jax.jit
def kernel(pattern, pattern_gt, mask):
    loss_fast, mismatch = _fast(pattern, pattern_gt, mask)
    return jax.lax.cond(
        mismatch == 0.0,
        lambda _: loss_fast,
        lambda _: _exact(pattern, pattern_gt, mask),
        operand=None,
    )


# speedup vs baseline: 4.7646x; 1.0003x over previous
import jax
import jax.numpy as jnp
from jax.experimental import pallas as pl
from jax.experimental.pallas import tpu as pltpu

_H = 384
_W = 384
_N = float(_H * _W)
_R = _H // 8
_CPB = 8
_NB = 96 // _CPB


def _fold3(a):
    return jnp.maximum(jnp.maximum(a[:, :128], a[:, 128:256]), a[:, 256:])


def _fold3s(a):
    return a[:, :128] + a[:, 128:256] + a[:, 256:]


def _vscalar(a):
    return jnp.sum(jnp.sum(a, axis=1, keepdims=True), axis=0,
                   keepdims=True)


def _vscalarw(a):
    return jnp.sum(jnp.sum(a, axis=1, keepdims=True), axis=0,
                   keepdims=True)


def _step(i, x_ref, m2_ref, m2g_ref, cef_ref, out_ref,
          xsA_ref, wsvA_ref, accA_ref, xsB_ref, wsvB_ref, accB_ref):
    def tau_step(_, carry):
        out = []
        for c in range(_CPB):
            Ma, Mb, tau, rem = carry[c]
            mx = jnp.maximum(jnp.max(Ma, axis=1, keepdims=True),
                             jnp.max(Mb, axis=1, keepdims=True))
            mx = jnp.max(mx, axis=0, keepdims=True)
            mxb = jax.lax.broadcast_in_dim(mx, (8, _W), (0, 1))
            hita = Ma == mxb
            hitb = Mb == mxb
            cnte = _vscalarw(jnp.where(hita, 1.0, 0.0)
                             + jnp.where(hitb, 1.0, 0.0))
            tau = jnp.where(rem > 0.0, mx, tau)
            rem = rem - cnte
            Ma = jnp.where(hita, -jnp.inf, Ma)
            Mb = jnp.where(hitb, -jnp.inf, Mb)
            out.append((Ma, Mb, tau, rem))
        return tuple(out)

    ten = jnp.full((1, 1), 10.0, jnp.float32)
    tau_state = tuple(
        (wsvB_ref[c, 0], wsvB_ref[c, 1],
         jnp.full((1, 1), -jnp.inf, jnp.float32), ten)
        for c in range(_CPB)
    )

    zero1 = jnp.zeros((8, 128), jnp.float32)
    neginf = jnp.full((8, _W), -jnp.inf, jnp.float32)
    mains = [(zero1, zero1, zero1, zero1, neginf, neginf)
             for _ in range(_CPB)]
    filts = [(zero1, zero1, zero1) for _ in range(_CPB)]
    tau_wide = None

    def main_chunk(j):
        m2c = m2_ref[0, j]
        m2gc = m2g_ref[0, j]
        for c in range(_CPB):
            aS, aA, aB, aD, aM, aM2 = mains[c]
            xv = x_ref[c, j]
            xsA_ref[c, j] = xv
            vm2 = xv * m2c
            aS = aS + _fold3s(xv)
            aA = aA + _fold3s(xv * vm2)
            aB = aB + _fold3s(vm2)
            aD = aD + _fold3s(xv * m2gc)
            mn = jnp.minimum(aM, xv)
            aM = jnp.maximum(aM, xv)
            aM2 = jnp.maximum(aM2, mn)
            mains[c] = (aS, aA, aB, aD, aM, aM2)

    def filt_chunk(j):
        for c in range(_CPB):
            aGT, aV, aEQ = filts[c]
            pv = xsB_ref[c, j]
            gt = pv > tau_wide[c]
            eq = pv == tau_wide[c]
            aGT = aGT + _fold3s(jnp.where(gt, 1.0, 0.0))
            aV = aV + _fold3s(jnp.where(gt, pv, 0.0))
            aEQ = aEQ + _fold3s(jnp.where(eq, 1.0, 0.0))
            filts[c] = (aGT, aV, aEQ)

    for j in range(_R):
        main_chunk(j)
        if j < 10:
            tau_state = tau_step(j, tau_state)
        if j == 10:
            tau_wide = [
                jax.lax.broadcast_in_dim(tau_state[c][2], (8, _W), (0, 1))
                for c in range(_CPB)
            ]
        if 10 <= j < 20:
            filt_chunk(2 * (j - 10))
            filt_chunk(2 * (j - 10) + 1)
        elif j >= 20:
            filt_chunk(j)

    taus_c = tau_state
    sums_prev = [
        (_vscalar(accB_ref[c, 0]), _vscalar(accB_ref[c, 1]),
         _vscalar(accB_ref[c, 2]), _vscalar(accB_ref[c, 3]))
        for c in range(_CPB)
    ]

    for c in range(_CPB):
        aS, aA, aB, aD, M, M2 = mains[c]
        wsvA_ref[c, 0] = M
        wsvA_ref[c, 1] = M2
        accA_ref[c, 0] = aS
        accA_ref[c, 1] = aA
        accA_ref[c, 2] = aB
        accA_ref[c, 3] = aD

    C = cef_ref[0, 0:1, 0:1]
    E = cef_ref[0, 1:2, 0:1]
    F = cef_ref[0, 2:3, 0:1]

    loss = jnp.zeros((1, 1), jnp.float32)
    mismatch = jnp.zeros((1, 1), jnp.float32)
    for c in range(_CPB):
        cnt_gt = _vscalar(filts[c][0])
        sum_gt = _vscalar(filts[c][1])
        cnt_eq = _vscalar(filts[c][2])
        tau = taus_c[c][2]
        top10 = sum_gt + (10.0 - cnt_gt) * tau
        dirty = (cnt_gt > 10.0) | ((cnt_gt + cnt_eq) < 10.0)
        mismatch = mismatch + jnp.where(dirty, 1.0, 0.0)
        S, A, B, D = sums_prev[c]
        mu = S / _N
        max_avg = top10 / 10.0 - mu
        denom = jnp.where(max_avg < 1e-20, max_avg + 1e-19, max_avg)
        num = ((A - 2.0 * mu * B + mu * mu * C) / denom
               - 2.0 * (D - mu * E)) / denom + F
        loss = loss + num / _N

    gate = jnp.where(i > 0, 1.0, 0.0)
    loss = jnp.where(gate > 0.0, loss, 0.0)
    mismatch = jnp.where(gate > 0.0, mismatch, 0.0)
    upd = jnp.concatenate([
        jax.lax.broadcast_in_dim(loss, (1, 8, 128), (1, 2)),
        jax.lax.broadcast_in_dim(mismatch, (1, 8, 128), (1, 2)),
    ], axis=0)
    out_ref[...] += upd


def _body(x_ref, gt_ref, m_ref, out_ref,
          m2_ref, m2g_ref, cef_ref,
          xs0_ref, xs1_ref, wsv0_ref, wsv1_ref, acc0_ref, acc1_ref):
    i = pl.program_id(0)
    par = jax.lax.rem(i, 2)

    @pl.when(i == 0)
    def _():
        m = m_ref[0]
        g = gt_ref[0]
        m2 = m * m
        m2g = m2 * g
        m2_ref[0] = m2
        m2g_ref[0] = m2g
        cef_ref[...] = jnp.concatenate([
            jax.lax.broadcast_in_dim(_vscalar2(m2), (1, 1, 128), (1, 2)),
            jax.lax.broadcast_in_dim(_vscalar2(m2g), (1, 1, 128), (1, 2)),
            jax.lax.broadcast_in_dim(_vscalar2(m2g * g), (1, 1, 128),
                                     (1, 2)),
        ], axis=1)
        out_ref[...] = jnp.zeros(out_ref.shape, jnp.float32)

    @pl.when(par == 0)
    def _():
        _step(i, x_ref, m2_ref, m2g_ref, cef_ref, out_ref,
              xs0_ref, wsv0_ref, acc0_ref, xs1_ref, wsv1_ref, acc1_ref)

    @pl.when(par == 1)
    def _():
        _step(i, x_ref, m2_ref, m2g_ref, cef_ref, out_ref,
              xs1_ref, wsv1_ref, acc1_ref, xs0_ref, wsv0_ref, acc0_ref)


def _vscalar2(a):
    s = jnp.sum(a, axis=0)
    s = _fold3s(s)
    return _vscalar(s)


def _fast(pattern, pattern_gt, mask):
    ch = pattern.shape[1]
    x = pattern.reshape(ch, _R, 8, _W)
    out = pl.pallas_call(
        _body,
        grid=(_NB + 1,),
        in_specs=[
            pl.BlockSpec((_CPB, _R, 8, _W),
                         lambda i: (jnp.minimum(i, _NB - 1), 0, 0, 0)),
            pl.BlockSpec((1, _R, 8, _W), lambda i: (0, 0, 0, 0)),
            pl.BlockSpec((1, _R, 8, _W), lambda i: (0, 0, 0, 0)),
        ],
        out_specs=pl.BlockSpec((2, 8, 128), lambda i: (0, 0, 0)),
        out_shape=jax.ShapeDtypeStruct((2, 8, 128), jnp.float32),
        scratch_shapes=[
            pltpu.VMEM((1, _R, 8, _W), jnp.float32),
            pltpu.VMEM((1, _R, 8, _W), jnp.float32),
            pltpu.VMEM((1, 3, 128), jnp.float32),
            pltpu.VMEM((_CPB, _R, 8, _W), jnp.float32),
            pltpu.VMEM((_CPB, _R, 8, _W), jnp.float32),
            pltpu.VMEM((_CPB, 2, 8, _W), jnp.float32),
            pltpu.VMEM((_CPB, 2, 8, _W), jnp.float32),
            pltpu.VMEM((_CPB, 4, 8, 128), jnp.float32),
            pltpu.VMEM((_CPB, 4, 8, 128), jnp.float32),
        ],
    )(x, pattern_gt.reshape(1, _R, 8, _W), mask.reshape(1, _R, 8, _W))
    return out[0, 0, 0].reshape(1), out[1, 0, 0]


def _exact_body(x_ref, gt_ref, m_ref, out_ref, m2_ref, m2g_ref, cef_ref):
    i = pl.program_id(0)

    @pl.when(i == 0)
    def _():
        m = m_ref[0]
        g = gt_ref[0]
        m2 = m * m
        m2g = m2 * g
        m2_ref[0] = m2
        m2g_ref[0] = m2g
        cef_ref[0] = jnp.sum(m2)
        cef_ref[1] = jnp.sum(m2g)
        cef_ref[2] = jnp.sum(m2g * g)
        out_ref[...] = jnp.zeros(out_ref.shape, jnp.float32)

    def chunk(j, carry):
        aS, aA, aB, aD, aM = carry
        xv = x_ref[0, j]
        m2c = m2_ref[0, j]
        m2gc = m2g_ref[0, j]
        vm2 = xv * m2c
        return (aS + xv, aA + xv * vm2, aB + vm2, aD + xv * m2gc,
                jnp.maximum(aM, xv))

    zero = jnp.zeros((8, _W), jnp.float32)
    aS, aA, aB, aD, M = jax.lax.fori_loop(
        0, _R, chunk,
        (zero, zero, zero, zero, jnp.full((8, _W), -jnp.inf, jnp.float32)),
        unroll=4,
    )
    S = jnp.sum(aS)
    A = jnp.sum(aA)
    B = jnp.sum(aB)
    D = jnp.sum(aD)

    def tau_step(_, carry):
        MM, tau = carry
        mx = jnp.max(MM)
        MM = jnp.where(MM == mx, -jnp.inf, MM)
        return MM, mx

    _, tau = jax.lax.fori_loop(0, 10, tau_step, (M, jnp.float32(-jnp.inf)))

    def fchunk(j, carry):
        aC, aV = carry
        xv = x_ref[0, j]
        sel = xv >= tau
        return (aC + jnp.where(sel, 1.0, 0.0),
                aV + jnp.where(sel, xv, 0.0))

    aC, aV = jax.lax.fori_loop(0, _R, fchunk, (zero, zero), unroll=4)
    cnt = jnp.sum(aC)
    ssum = jnp.sum(aV)

    def exact_fallback(_):
        def step(_, carry):
            bound, acc, rem = carry
            v = x_ref[0]
            w = jnp.where((v >= tau) & (v < bound), v, -jnp.inf)
            mx = jnp.max(w)
            cc = jnp.sum(jnp.where(w == mx, 1.0, 0.0))
            take = jnp.minimum(cc, rem)
            acc = acc + jnp.where(take > 0.0, take * mx, 0.0)
            rem = rem - take
            return mx, acc, rem

        _, acc, _ = jax.lax.fori_loop(
            0, 10, step, (jnp.float32(jnp.inf), jnp.float32(0.0),
                          jnp.float32(10.0))
        )
        return acc

    top10_sum = jax.lax.cond(cnt == 10.0, lambda _: ssum, exact_fallback,
                             operand=None)

    C = cef_ref[0]
    E = cef_ref[1]
    F = cef_ref[2]

    mu = S / _N
    max_avg = top10_sum / 10.0 - mu
    denom = jnp.where(max_avg < 1e-20, max_avg + 1e-19, max_avg)
    num = ((A - 2.0 * mu * B + mu * mu * C) / denom
           - 2.0 * (D - mu * E)) / denom + F
    loss_i = num / _N

    out_ref[...] += jnp.full(out_ref.shape, loss_i, dtype=jnp.float32)


def _exact(pattern, pattern_gt, mask):
    ch = pattern.shape[1]
    x = pattern.reshape(ch, _R, 8, _W)
    out = pl.pallas_call(
        _exact_body,
        grid=(ch,),
        in_specs=[
            pl.BlockSpec((1, _R, 8, _W), lambda i: (i, 0, 0, 0)),
            pl.BlockSpec((1, _R, 8, _W), lambda i: (0, 0, 0, 0)),
            pl.BlockSpec((1, _R, 8, _W), lambda i: (0, 0, 0, 0)),
        ],
        out_specs=pl.BlockSpec((8, 128), lambda i: (0, 0)),
        out_shape=jax.ShapeDtypeStruct((8, 128), jnp.float32),
        scratch_shapes=[
            pltpu.VMEM((1, _R, 8, _W), jnp.float32),
            pltpu.VMEM((1, _R, 8, _W), jnp.float32),
            pltpu.SMEM((3,), jnp.float32),
        ],
    )(x, pattern_gt.reshape(1, _R, 8, _W), mask.reshape(1, _R, 8, _W))
    return out[0, 0].reshape(1)


@jax.jit
def kernel(pattern, pattern_gt, mask):
    loss_fast, mismatch = _fast(pattern, pattern_gt, mask)
    return jax.lax.cond(
        mismatch == 0.0,
        lambda _: loss_fast,
        lambda _: _exact(pattern, pattern_gt, mask),
        operand=None,
    )
